# baseline (speedup 1.0000x reference)
import jax
import jax.numpy as jnp
from jax import lax
from jax.experimental import pallas as pl
from jax.experimental.pallas import tpu as pltpu
from jax.experimental.pallas import tpu_sc as plsc

N = 100000
M = 16384
D = 256
NC, NS = 2, 16
NW = NC * NS
NR = 3128
CPR = 512
K = 64
POSB = 14
SENT = 1 << 30
NTC = 196

_i32 = jnp.int32


def _take16(x, idx):
    dnums = lax.GatherDimensionNumbers(
        offset_dims=(), collapsed_slice_dims=(0,), start_index_map=(0,))
    return lax.gather(x, idx[:, None], dnums, slice_sizes=(1,),
                      mode=lax.GatherScatterMode.PROMISE_IN_BOUNDS)


def _tc_copy_body(in_ref, out_ref):
    out_ref[...] = in_ref[...]


def _sc_body(out_hbm, idx_hbm, upd_hbm,
             gb0, gb1, ixr, cnts, tbl, wsrc, wdst,
             sidx0, sidx1, didx0, didx1,
             g0, g1, s0, s1, isem):
    wid = lax.axis_index("s") * NC + lax.axis_index("c")
    base = wid * NR
    limit = jnp.minimum(base + NR, N)
    nrw = limit - base
    iota = lax.iota(_i32, 16)

    pltpu.async_copy(idx_hbm, ixr, isem)
    pltpu.make_async_copy(idx_hbm, ixr, isem).wait()

    def scan_body(c, _):
        v = ixr[pl.ds(c * 16, 16)]
        m = (v >= base) & (v < limit)
        code = jnp.where(m, (v - base) * (1 << POSB) + (c * 16 + iota), SENT)
        ixr[pl.ds(c * 16, 16)] = code
        f = jnp.where(m, 1, 0)
        for s in (1, 2, 4, 8):
            f = f | _take16(f, iota ^ s)
        cnts[pl.ds(c * 16, 16)] = f
        return _

    lax.fori_loop(0, M // 16, scan_body, jnp.int32(0))

    def init_body(c, _):
        tbl[pl.ds(c * 16, 16)] = jnp.full((16,), -1, _i32)
        return _

    lax.fori_loop(0, NTC, init_body, jnp.int32(0))

    def replay_body(c, _):
        flag = cnts[pl.ds(c * 16, 16)][0]

        @pl.when(flag > 0)
        def _chunk():
            codes = ixr[pl.ds(c * 16, 16)]
            for l in range(16):
                code = codes[l]

                @pl.when(code < SENT)
                def _hit(code=code):
                    r = code >> POSB
                    pos = code & ((1 << POSB) - 1)
                    r_al = pl.multiple_of((r >> 4) * 16, 16)
                    lane = r & 15
                    w = tbl[pl.ds(r_al, 16)]
                    tbl[pl.ds(r_al, 16)] = jnp.where(iota == lane, pos, w)
        return _

    lax.fori_loop(0, M // 16, replay_body, jnp.int32(0))

    def win_body(c, carry):
        tv = tbl[pl.ds(c * 16, 16)]

        def lane_step(l, carry):
            w, lp, ld = carry
            pos = tv[l]
            row = c * 16 + l

            def emit(_):
                w_al = pl.multiple_of((w >> 4) * 16, 16)
                lane = w & 15
                sv = wsrc[pl.ds(w_al, 16)]
                wsrc[pl.ds(w_al, 16)] = jnp.where(iota == lane, pos, sv)
                dv = wdst[pl.ds(w_al, 16)]
                wdst[pl.ds(w_al, 16)] = jnp.where(iota == lane, base + row, dv)
                return (w + 1, pos, base + row)
            return lax.cond((pos >= 0) & (row < nrw), emit,
                            lambda _: carry, 0)

        for l in range(16):
            carry = lane_step(l, carry)
        return carry

    wcnt, lastp, lastd = lax.fori_loop(
        0, NTC, win_body, (jnp.int32(0), jnp.int32(0), jnp.int32(0)))

    @pl.when(wcnt > 0)
    def _pad():
        a0 = pl.multiple_of((wcnt >> 4) * 16, 16)
        ps = jnp.full((16,), 0, _i32) + lastp
        pd = jnp.full((16,), 0, _i32) + lastd
        sv = wsrc[pl.ds(a0, 16)]
        dv = wdst[pl.ds(a0, 16)]
        keep = iota < (wcnt - a0)
        wsrc[pl.ds(a0, 16)] = jnp.where(keep, sv, ps)
        wdst[pl.ds(a0, 16)] = jnp.where(keep, dv, pd)
        for j in range(1, 1 + K // 16):
            wsrc[pl.ds(a0 + j * 16, 16)] = ps
            wdst[pl.ds(a0 + j * 16, 16)] = pd

    nbat = (wcnt + K - 1) // K
    sbufs = (gb0, gb1)
    sidx, didx = (sidx0, sidx1), (didx0, didx1)
    gsem, ssem = (g0, g1), (s0, s1)

    def bat_body(g, _):
        for p in range(2):
            bat = g * 2 + p

            @pl.when(bat < nbat)
            def _do(p=p, bat=bat):
                @pl.when(bat >= 2)
                def _wait_prev():
                    pltpu.make_async_copy(sbufs[p], out_hbm.at[didx[p]],
                                          ssem[p]).wait()

                for k2 in range(K // 16):
                    sl = pl.ds(bat * K + k2 * 16, 16)
                    sidx[p][pl.ds(k2 * 16, 16)] = wsrc[sl]
                    didx[p][pl.ds(k2 * 16, 16)] = wdst[sl]
                pltpu.async_copy(upd_hbm.at[sidx[p]], sbufs[p], gsem[p])
                pltpu.make_async_copy(upd_hbm.at[sidx[p]], sbufs[p],
                                      gsem[p]).wait()
                pltpu.async_copy(sbufs[p], out_hbm.at[didx[p]], ssem[p])
        return _

    lax.fori_loop(0, (nbat + 1) // 2, bat_body, jnp.int32(0))

    for p in range(2):
        @pl.when(nbat > p)
        def _drain(p=p):
            pltpu.make_async_copy(sbufs[p], out_hbm.at[didx[p]],
                                  ssem[p]).wait()


def _tpose_body(in_ref, out_ref):
    out_ref[...] = in_ref[...].T


@jax.jit
def _scatter_overwrite(input, indices, update):
    in_nat = jnp.transpose(input, (1, 2, 0)).reshape(D, N)
    upd_nat = jnp.transpose(update, (1, 2, 0)).reshape(D, M)

    copied = pl.pallas_call(
        _tpose_body,
        out_shape=jax.ShapeDtypeStruct((N, D), jnp.float32),
        grid=(pl.cdiv(N, CPR),),
        in_specs=[pl.BlockSpec((D, CPR), lambda i: (0, i))],
        out_specs=pl.BlockSpec((CPR, D), lambda i: (i, 0)),
    )(in_nat)

    upd2d = pl.pallas_call(
        _tpose_body,
        out_shape=jax.ShapeDtypeStruct((M, D), jnp.float32),
        grid=(M // D,),
        in_specs=[pl.BlockSpec((D, D), lambda i: (0, i))],
        out_specs=pl.BlockSpec((D, D), lambda i: (i, 0)),
    )(upd_nat)

    mesh = plsc.VectorSubcoreMesh(core_axis_name="c", subcore_axis_name="s")
    sc_f = pl.kernel(
        _sc_body,
        out_type=(),
        mesh=mesh,
        scratch_types=[
            pltpu.VMEM((K, D), jnp.float32),
            pltpu.VMEM((K, D), jnp.float32),
            pltpu.VMEM((M,), _i32),
            pltpu.VMEM((M,), _i32),
            pltpu.VMEM((NTC * 16,), _i32),
            pltpu.VMEM((NR + 2 * K,), _i32),
            pltpu.VMEM((NR + 2 * K,), _i32),
            pltpu.VMEM((K,), _i32),
            pltpu.VMEM((K,), _i32),
            pltpu.VMEM((K,), _i32),
            pltpu.VMEM((K,), _i32),
        ] + [pltpu.SemaphoreType.DMA] * 5,
    )
    ref = jax.new_ref(copied)
    sc_f(ref, indices, upd2d)
    scattered = ref[...]

    out_nat = pl.pallas_call(
        _tpose_body,
        out_shape=jax.ShapeDtypeStruct((D, N), jnp.float32),
        grid=(pl.cdiv(N, CPR),),
        in_specs=[pl.BlockSpec((CPR, D), lambda i: (i, 0))],
        out_specs=pl.BlockSpec((D, CPR), lambda i: (0, i)),
    )(scattered)
    return jnp.transpose(out_nat.reshape(4, 64, N), (2, 0, 1))


def kernel(input, indices, update):
    return _scatter_overwrite(input, indices.astype(jnp.int32), update)


# speedup vs baseline: 1.4891x; 1.4891x over previous
import jax
import jax.numpy as jnp
from jax import lax
from jax.experimental import pallas as pl
from jax.experimental.pallas import tpu as pltpu
from jax.experimental.pallas import tpu_sc as plsc

N = 100000
M = 16384
D = 256
NC, NS = 2, 16
NW = NC * NS
NR = 3128
CPR = 512
K = 64
POSB = 14
SENT = 1 << 30
NTC = 196

_i32 = jnp.int32


def _take16(x, idx):
    dnums = lax.GatherDimensionNumbers(
        offset_dims=(), collapsed_slice_dims=(0,), start_index_map=(0,))
    return lax.gather(x, idx[:, None], dnums, slice_sizes=(1,),
                      mode=lax.GatherScatterMode.PROMISE_IN_BOUNDS)


def _tc_copy_body(in_ref, out_ref):
    out_ref[...] = in_ref[...]


def _sc_body(out_hbm, idx_hbm, upd_hbm,
             gb0, gb1, ixr, cnts, tbl, wsrc, wdst,
             sidx0, sidx1, didx0, didx1,
             g0, g1, s0, s1, isem):
    wid = lax.axis_index("s") * NC + lax.axis_index("c")
    base = wid * NR
    limit = jnp.minimum(base + NR, N)
    nrw = limit - base
    iota = lax.iota(_i32, 16)

    pltpu.async_copy(idx_hbm, ixr, isem)
    pltpu.make_async_copy(idx_hbm, ixr, isem).wait()

    def scan_body(c, _):
        v = ixr[pl.ds(c * 16, 16)]
        m = (v >= base) & (v < limit)
        code = jnp.where(m, (v - base) * (1 << POSB) + (c * 16 + iota), SENT)
        ixr[pl.ds(c * 16, 16)] = code
        f = jnp.where(m, 1, 0)
        for s in (1, 2, 4, 8):
            f = f | _take16(f, iota ^ s)
        cnts[pl.ds(c * 16, 16)] = f
        return _

    lax.fori_loop(0, M // 16, scan_body, jnp.int32(0))

    def init_body(c, _):
        tbl[pl.ds(c * 16, 16)] = jnp.full((16,), -1, _i32)
        return _

    lax.fori_loop(0, NTC, init_body, jnp.int32(0))

    def replay_body(c, _):
        flag = cnts[pl.ds(c * 16, 16)][0]

        @pl.when(flag > 0)
        def _chunk():
            codes = ixr[pl.ds(c * 16, 16)]
            for l in range(16):
                code = codes[l]

                @pl.when(code < SENT)
                def _hit(code=code):
                    r = code >> POSB
                    pos = code & ((1 << POSB) - 1)
                    r_al = pl.multiple_of((r >> 4) * 16, 16)
                    lane = r & 15
                    w = tbl[pl.ds(r_al, 16)]
                    tbl[pl.ds(r_al, 16)] = jnp.where(iota == lane, pos, w)
        return _

    lax.fori_loop(0, M // 16, replay_body, jnp.int32(0))

    def win_body(c, carry):
        tv = tbl[pl.ds(c * 16, 16)]

        def lane_step(l, carry):
            w, lp, ld = carry
            pos = tv[l]
            row = c * 16 + l

            def emit(_):
                w_al = pl.multiple_of((w >> 4) * 16, 16)
                lane = w & 15
                sv = wsrc[pl.ds(w_al, 16)]
                wsrc[pl.ds(w_al, 16)] = jnp.where(iota == lane, pos, sv)
                dv = wdst[pl.ds(w_al, 16)]
                wdst[pl.ds(w_al, 16)] = jnp.where(iota == lane, base + row, dv)
                return (w + 1, pos, base + row)
            return lax.cond((pos >= 0) & (row < nrw), emit,
                            lambda _: carry, 0)

        for l in range(16):
            carry = lane_step(l, carry)
        return carry

    wcnt, lastp, lastd = lax.fori_loop(
        0, NTC, win_body, (jnp.int32(0), jnp.int32(0), jnp.int32(0)))

    @pl.when(wcnt > 0)
    def _pad():
        a0 = pl.multiple_of((wcnt >> 4) * 16, 16)
        ps = jnp.full((16,), 0, _i32) + lastp
        pd = jnp.full((16,), 0, _i32) + lastd
        sv = wsrc[pl.ds(a0, 16)]
        dv = wdst[pl.ds(a0, 16)]
        keep = iota < (wcnt - a0)
        wsrc[pl.ds(a0, 16)] = jnp.where(keep, sv, ps)
        wdst[pl.ds(a0, 16)] = jnp.where(keep, dv, pd)
        for j in range(1, 1 + K // 16):
            wsrc[pl.ds(a0 + j * 16, 16)] = ps
            wdst[pl.ds(a0 + j * 16, 16)] = pd

    nbat = (wcnt + K - 1) // K
    sbufs = (gb0, gb1)
    sidx, didx = (sidx0, sidx1), (didx0, didx1)
    gsem, ssem = (g0, g1), (s0, s1)

    def bat_body(g, _):
        for p in range(2):
            bat = g * 2 + p

            @pl.when(bat < nbat)
            def _do(p=p, bat=bat):
                @pl.when(bat >= 2)
                def _wait_prev():
                    pltpu.make_async_copy(sbufs[p], out_hbm.at[didx[p]],
                                          ssem[p]).wait()

                for k2 in range(K // 16):
                    sl = pl.ds(bat * K + k2 * 16, 16)
                    sidx[p][pl.ds(k2 * 16, 16)] = wsrc[sl]
                    didx[p][pl.ds(k2 * 16, 16)] = wdst[sl]
                pltpu.async_copy(upd_hbm.at[sidx[p]], sbufs[p], gsem[p])
                pltpu.make_async_copy(upd_hbm.at[sidx[p]], sbufs[p],
                                      gsem[p]).wait()
                pltpu.async_copy(sbufs[p], out_hbm.at[didx[p]], ssem[p])
        return _

    lax.fori_loop(0, (nbat + 1) // 2, bat_body, jnp.int32(0))

    for p in range(2):
        @pl.when(nbat > p)
        def _drain(p=p):
            pltpu.make_async_copy(sbufs[p], out_hbm.at[didx[p]],
                                  ssem[p]).wait()


@jax.jit
def _scatter_overwrite(input, indices, update):
    copied = input.reshape(N, D)
    upd2d = update.reshape(M, D)

    mesh = plsc.VectorSubcoreMesh(core_axis_name="c", subcore_axis_name="s")
    sc_f = pl.kernel(
        _sc_body,
        out_type=(),
        mesh=mesh,
        scratch_types=[
            pltpu.VMEM((K, D), jnp.float32),
            pltpu.VMEM((K, D), jnp.float32),
            pltpu.VMEM((M,), _i32),
            pltpu.VMEM((M,), _i32),
            pltpu.VMEM((NTC * 16,), _i32),
            pltpu.VMEM((NR + 2 * K,), _i32),
            pltpu.VMEM((NR + 2 * K,), _i32),
            pltpu.VMEM((K,), _i32),
            pltpu.VMEM((K,), _i32),
            pltpu.VMEM((K,), _i32),
            pltpu.VMEM((K,), _i32),
        ] + [pltpu.SemaphoreType.DMA] * 5,
    )
    ref = jax.new_ref(copied)
    sc_f(ref, indices, upd2d)
    return ref[...].reshape(N, 4, 64)


def kernel(input, indices, update):
    return _scatter_overwrite(input, indices.astype(jnp.int32), update)


# speedup vs baseline: 2.0102x; 1.3500x over previous
import jax
import jax.numpy as jnp
from jax import lax
from jax.experimental import pallas as pl
from jax.experimental.pallas import tpu as pltpu
from jax.experimental.pallas import tpu_sc as plsc

N = 100000
M = 16384
D = 256
NC, NS = 2, 16
NW = NC * NS
NR = 3128
K = 64
POSB = 14
SENT = 1 << 30
NTC = 196
WL = NR + 2 * K

_i32 = jnp.int32


def _take16(x, idx):
    dnums = lax.GatherDimensionNumbers(
        offset_dims=(), collapsed_slice_dims=(0,), start_index_map=(0,))
    return lax.gather(x, idx[:, None], dnums, slice_sizes=(1,),
                      mode=lax.GatherScatterMode.PROMISE_IN_BOUNDS)


def _sca_body(idx_hbm, wsrc_hbm, wdst_hbm,
              ixr, cnts, tbl, wsrc, wdst, isem, osem):
    wid = lax.axis_index("s") * NC + lax.axis_index("c")
    base = wid * NR
    limit = jnp.minimum(base + NR, N)
    nrw = limit - base
    iota = lax.iota(_i32, 16)

    pltpu.async_copy(idx_hbm, ixr, isem)
    pltpu.make_async_copy(idx_hbm, ixr, isem).wait()

    def scan_body(c, _):
        v = ixr[pl.ds(c * 16, 16)]
        m = (v >= base) & (v < limit)
        code = jnp.where(m, (v - base) * (1 << POSB) + (c * 16 + iota), SENT)
        ixr[pl.ds(c * 16, 16)] = code
        f = jnp.where(m, 1, 0)
        for s in (1, 2, 4, 8):
            f = f | _take16(f, iota ^ s)
        cnts[pl.ds(c * 16, 16)] = f
        return _

    lax.fori_loop(0, M // 16, scan_body, jnp.int32(0))

    def init_body(c, _):
        tbl[pl.ds(c * 16, 16)] = jnp.full((16,), -1, _i32)
        return _

    lax.fori_loop(0, NTC, init_body, jnp.int32(0))

    def replay_body(c, _):
        flag = cnts[pl.ds(c * 16, 16)][0]

        @pl.when(flag > 0)
        def _chunk():
            codes = ixr[pl.ds(c * 16, 16)]
            for l in range(16):
                code = codes[l]

                @pl.when(code < SENT)
                def _hit(code=code):
                    r = code >> POSB
                    pos = code & ((1 << POSB) - 1)
                    r_al = pl.multiple_of((r >> 4) * 16, 16)
                    lane = r & 15
                    w = tbl[pl.ds(r_al, 16)]
                    tbl[pl.ds(r_al, 16)] = jnp.where(iota == lane, pos, w)
        return _

    lax.fori_loop(0, M // 16, replay_body, jnp.int32(0))

    def win_body(c, carry):
        tv = tbl[pl.ds(c * 16, 16)]

        def lane_step(l, carry):
            w, lp, ld = carry
            pos = tv[l]
            row = c * 16 + l

            def emit(_):
                w_al = pl.multiple_of((w >> 4) * 16, 16)
                lane = w & 15
                sv = wsrc[pl.ds(w_al, 16)]
                wsrc[pl.ds(w_al, 16)] = jnp.where(iota == lane, pos, sv)
                dv = wdst[pl.ds(w_al, 16)]
                wdst[pl.ds(w_al, 16)] = jnp.where(iota == lane, base + row, dv)
                return (w + 1, pos, base + row)
            return lax.cond((pos >= 0) & (row < nrw), emit,
                            lambda _: carry, 0)

        for l in range(16):
            carry = lane_step(l, carry)
        return carry

    wcnt, lastp, lastd = lax.fori_loop(
        0, NTC, win_body, (jnp.int32(0), jnp.int32(0), jnp.int32(0)))

    @pl.when(wcnt > 0)
    def _pad():
        a0 = pl.multiple_of((wcnt >> 4) * 16, 16)
        ps = jnp.full((16,), 0, _i32) + lastp
        pd = jnp.full((16,), 0, _i32) + lastd
        sv = wsrc[pl.ds(a0, 16)]
        dv = wdst[pl.ds(a0, 16)]
        keep = iota < (wcnt - a0)
        wsrc[pl.ds(a0, 16)] = jnp.where(keep, sv, ps)
        wdst[pl.ds(a0, 16)] = jnp.where(keep, dv, pd)
        for j in range(1, 1 + K // 16):
            wsrc[pl.ds(a0 + j * 16, 16)] = ps
            wdst[pl.ds(a0 + j * 16, 16)] = pd

    wsrc[pl.ds(WL - 16, 16)] = jnp.full((16,), 0, _i32) + wcnt
    wb = pl.multiple_of(wid * WL, 8)
    pltpu.async_copy(wsrc, wsrc_hbm.at[pl.ds(wb, WL)], osem)
    pltpu.make_async_copy(wsrc, wsrc_hbm.at[pl.ds(wb, WL)], osem).wait()
    pltpu.async_copy(wdst, wdst_hbm.at[pl.ds(wb, WL)], osem)
    pltpu.make_async_copy(wdst, wdst_hbm.at[pl.ds(wb, WL)], osem).wait()


def _scb_body(out_hbm, upd_hbm, wsrc_hbm, wdst_hbm,
              gb0, gb1, wsrc, wdst,
              sidx0, sidx1, didx0, didx1,
              g0, g1, s0, s1, isem):
    wid = lax.axis_index("s") * NC + lax.axis_index("c")

    wb = pl.multiple_of(wid * WL, 8)
    pltpu.async_copy(wsrc_hbm.at[pl.ds(wb, WL)], wsrc, isem)
    pltpu.make_async_copy(wsrc_hbm.at[pl.ds(wb, WL)], wsrc, isem).wait()
    pltpu.async_copy(wdst_hbm.at[pl.ds(wb, WL)], wdst, isem)
    pltpu.make_async_copy(wdst_hbm.at[pl.ds(wb, WL)], wdst, isem).wait()
    wcnt = wsrc[pl.ds(WL - 16, 16)][0]

    nbat = (wcnt + K - 1) // K
    sbufs = (gb0, gb1)
    sidx, didx = (sidx0, sidx1), (didx0, didx1)
    gsem, ssem = (g0, g1), (s0, s1)

    def bat_body(g, _):
        for p in range(2):
            bat = g * 2 + p

            @pl.when(bat < nbat)
            def _do(p=p, bat=bat):
                @pl.when(bat >= 2)
                def _wait_prev():
                    pltpu.make_async_copy(sbufs[p], out_hbm.at[didx[p]],
                                          ssem[p]).wait()

                for k2 in range(K // 16):
                    sl = pl.ds(bat * K + k2 * 16, 16)
                    sidx[p][pl.ds(k2 * 16, 16)] = wsrc[sl]
                    didx[p][pl.ds(k2 * 16, 16)] = wdst[sl]
                pltpu.async_copy(upd_hbm.at[sidx[p]], sbufs[p], gsem[p])
                pltpu.make_async_copy(upd_hbm.at[sidx[p]], sbufs[p],
                                      gsem[p]).wait()
                pltpu.async_copy(sbufs[p], out_hbm.at[didx[p]], ssem[p])
        return _

    lax.fori_loop(0, (nbat + 1) // 2, bat_body, jnp.int32(0))

    for p in range(2):
        @pl.when(nbat > p)
        def _drain(p=p):
            pltpu.make_async_copy(sbufs[p], out_hbm.at[didx[p]],
                                  ssem[p]).wait()


@jax.jit
def _scatter_overwrite(input, indices, update):
    mesh = plsc.VectorSubcoreMesh(core_axis_name="c", subcore_axis_name="s")

    sca = pl.kernel(
        _sca_body,
        out_type=(jax.ShapeDtypeStruct((NW * WL,), _i32),
                  jax.ShapeDtypeStruct((NW * WL,), _i32)),
        mesh=mesh,
        scratch_types=[
            pltpu.VMEM((M,), _i32),
            pltpu.VMEM((M,), _i32),
            pltpu.VMEM((NTC * 16,), _i32),
            pltpu.VMEM((WL,), _i32),
            pltpu.VMEM((WL,), _i32),
            pltpu.SemaphoreType.DMA,
            pltpu.SemaphoreType.DMA,
        ],
    )
    wsrc_h, wdst_h = sca(indices)

    copied = input.reshape(N, D)
    upd2d = update.reshape(M, D)

    scb = pl.kernel(
        _scb_body,
        out_type=(),
        mesh=mesh,
        scratch_types=[
            pltpu.VMEM((K, D), jnp.float32),
            pltpu.VMEM((K, D), jnp.float32),
            pltpu.VMEM((WL,), _i32),
            pltpu.VMEM((WL,), _i32),
            pltpu.VMEM((K,), _i32),
            pltpu.VMEM((K,), _i32),
            pltpu.VMEM((K,), _i32),
            pltpu.VMEM((K,), _i32),
        ] + [pltpu.SemaphoreType.DMA] * 5,
    )
    ref = jax.new_ref(copied)
    scb(ref, upd2d, wsrc_h, wdst_h)
    return ref[...].reshape(N, 4, 64)


def kernel(input, indices, update):
    return _scatter_overwrite(input, indices.astype(jnp.int32), update)
